# baseline (speedup 1.0000x reference)
import functools

import jax
import jax.numpy as jnp
import numpy as np
from jax import lax
from jax.experimental import pallas as pl
from jax.experimental.pallas import tpu as pltpu
from jax.experimental.pallas import tpu_sc as plsc

NUM_EMB = 1_000_000
DIM = 32
ROWS = 16384
COLS = 26
B_TOTAL = ROWS * COLS
NC = 2
NS = 16
NW = NC * NS
B_PER_W = B_TOTAL // NW
CHUNK = 128
N_CHUNKS = B_PER_W // CHUNK
NBUF = 8
N_GROUPS = N_CHUNKS // NBUF
OUT_SUBROWS = ROWS * 32 * 128 // DIM


def _emb_body(idx_hbm, qidx_hbm, table_hbm, out_hbm, idx_v, qidx_v, rows_v,
              *sems):
    gsems = sems[:NBUF]
    wsems = sems[NBUF:]
    wid = lax.axis_index("s") * NC + lax.axis_index("c")

    pltpu.sync_copy(idx_hbm.at[wid], idx_v)
    pltpu.sync_copy(qidx_hbm.at[wid], qidx_v)

    def gather(j, b):
        pltpu.make_async_copy(
            table_hbm.at[idx_v.at[j]], rows_v.at[b], gsems[b]
        ).start()

    def gather_wait(j, b):
        pltpu.make_async_copy(
            table_hbm.at[idx_v.at[j]], rows_v.at[b], gsems[b]
        ).wait()

    def write(j, b):
        pltpu.make_async_copy(
            rows_v.at[b], out_hbm.at[qidx_v.at[j]], wsems[b]
        ).start()

    def write_wait(j, b):
        pltpu.make_async_copy(
            rows_v.at[b], out_hbm.at[qidx_v.at[j]], wsems[b]
        ).wait()

    for b in range(NBUF - 1):
        gather(b, b)

    def body(g, carry):
        j0 = g * NBUF
        for b in range(NBUF):
            j = j0 + b
            gather_wait(j, b)
            write(j, b)
            bp = (b - 1) % NBUF
            jn = j + NBUF - 1

            if b == 0:
                @pl.when(j >= 1)
                def _():
                    write_wait(j - 1, bp)

                gather(jn, bp)
            else:
                @pl.when(jn < N_CHUNKS)
                def _():
                    write_wait(j - 1, bp)
                    gather(jn, bp)

        return carry

    lax.fori_loop(0, N_GROUPS, body, 0)

    for b in range(NBUF):
        write_wait(N_CHUNKS - NBUF + b, b)


_T = np.arange(B_TOTAL, dtype=np.int64)
_QIDX3 = ((_T // COLS) * 128 + (_T % COLS) * (128 // DIM)).astype(
    np.int32).reshape(NW, N_CHUNKS, CHUNK)


def _slice_body(src_ref, out_ref):
    out_ref[...] = src_ref[:, :COLS, :DIM]


def kernel(input, weight):
    idx = input.reshape(-1).astype(jnp.int32)
    idx3 = idx.reshape(NW, N_CHUNKS, CHUNK)
    qidx3 = jnp.asarray(_QIDX3)

    mesh = plsc.VectorSubcoreMesh(core_axis_name="c", subcore_axis_name="s")
    run = pl.kernel(
        _emb_body,
        out_type=jax.ShapeDtypeStruct((OUT_SUBROWS, DIM), jnp.float32),
        mesh=mesh,
        scratch_types=[
            pltpu.VMEM((N_CHUNKS, CHUNK), jnp.int32),
            pltpu.VMEM((N_CHUNKS, CHUNK), jnp.int32),
            pltpu.VMEM((NBUF, CHUNK, DIM), jnp.float32),
        ]
        + [pltpu.SemaphoreType.DMA] * (2 * NBUF),
        compiler_params=pltpu.CompilerParams(use_tc_tiling_on_sc=False),
    )
    out = run(idx3, qidx3, weight)
    dense3 = out.reshape(ROWS, 32, 128)
    BI = 128
    return pl.pallas_call(
        _slice_body,
        grid=(ROWS // BI,),
        in_specs=[pl.BlockSpec((BI, 32, 128), lambda i: (i, 0, 0))],
        out_specs=pl.BlockSpec((BI, COLS, DIM), lambda i: (i, 0, 0)),
        out_shape=jax.ShapeDtypeStruct((ROWS, COLS, DIM), jnp.float32),
    )(dense3)


# speedup vs baseline: 1.4542x; 1.4542x over previous
import functools

import jax
import jax.numpy as jnp
import numpy as np
from jax import lax
from jax.experimental import pallas as pl
from jax.experimental.pallas import tpu as pltpu
from jax.experimental.pallas import tpu_sc as plsc

NUM_EMB = 1_000_000
DIM = 32
ROWS = 16384
COLS = 26
B_TOTAL = ROWS * COLS
NC = 2
NS = 16
NW = NC * NS
B_PER_W = B_TOTAL // NW
CHUNK = 128
N_CHUNKS = B_PER_W // CHUNK
NBUF = 8
N_GROUPS = N_CHUNKS // NBUF
OUT_SUBROWS = ROWS * 32 * 128 // DIM


def _emb_body(idx_hbm, qidx_hbm, table_hbm, out_hbm, idx_v, qidx_v, rows_v,
              *sems):
    gsems = sems[:NBUF]
    wsems = sems[NBUF:]
    wid = lax.axis_index("s") * NC + lax.axis_index("c")

    pltpu.sync_copy(idx_hbm.at[wid], idx_v)
    pltpu.sync_copy(qidx_hbm.at[wid], qidx_v)

    def gather(j, b):
        pltpu.make_async_copy(
            table_hbm.at[idx_v.at[j]], rows_v.at[b], gsems[b]
        ).start()

    def gather_wait(j, b):
        pltpu.make_async_copy(
            table_hbm.at[idx_v.at[j]], rows_v.at[b], gsems[b]
        ).wait()

    def write(j, b):
        pltpu.make_async_copy(
            rows_v.at[b], out_hbm.at[qidx_v.at[j]], wsems[b]
        ).start()

    def write_wait(j, b):
        pltpu.make_async_copy(
            rows_v.at[b], out_hbm.at[qidx_v.at[j]], wsems[b]
        ).wait()

    for b in range(NBUF - 1):
        gather(b, b)

    def body(g, carry):
        j0 = g * NBUF
        for b in range(NBUF):
            j = j0 + b
            gather_wait(j, b)
            write(j, b)
            bp = (b - 1) % NBUF
            jn = j + NBUF - 1

            if b == 0:
                @pl.when(j >= 1)
                def _():
                    write_wait(j - 1, bp)

                gather(jn, bp)
            else:
                @pl.when(jn < N_CHUNKS)
                def _():
                    write_wait(j - 1, bp)
                    gather(jn, bp)

        return carry

    lax.fori_loop(0, N_GROUPS, body, 0)

    for b in range(NBUF):
        write_wait(N_CHUNKS - NBUF + b, b)


_T = np.arange(B_TOTAL, dtype=np.int64)
_QIDX3 = ((_T // COLS) * 128 + (_T % COLS) * (128 // DIM)).astype(
    np.int32).reshape(NW, N_CHUNKS, CHUNK)


def kernel(input, weight):
    idx = input.reshape(-1).astype(jnp.int32)
    idx3 = idx.reshape(NW, N_CHUNKS, CHUNK)
    qidx3 = jnp.asarray(_QIDX3)

    mesh = plsc.VectorSubcoreMesh(core_axis_name="c", subcore_axis_name="s")
    run = pl.kernel(
        _emb_body,
        out_type=jax.ShapeDtypeStruct((OUT_SUBROWS, DIM), jnp.float32),
        mesh=mesh,
        scratch_types=[
            pltpu.VMEM((N_CHUNKS, CHUNK), jnp.int32),
            pltpu.VMEM((N_CHUNKS, CHUNK), jnp.int32),
            pltpu.VMEM((NBUF, CHUNK, DIM), jnp.float32),
        ]
        + [pltpu.SemaphoreType.DMA] * (2 * NBUF),
        compiler_params=pltpu.CompilerParams(use_tc_tiling_on_sc=False),
    )
    out = run(idx3, qidx3, weight)
    return out.reshape(ROWS, 32, 128)[:, :COLS, :DIM]


# speedup vs baseline: 1.4548x; 1.0005x over previous
import functools

import jax
import jax.numpy as jnp
import numpy as np
from jax import lax
from jax.experimental import pallas as pl
from jax.experimental.pallas import tpu as pltpu
from jax.experimental.pallas import tpu_sc as plsc

NUM_EMB = 1_000_000
DIM = 32
ROWS = 16384
COLS = 26
B_TOTAL = ROWS * COLS
NC = 2
NS = 16
NW = NC * NS
B_PER_W = B_TOTAL // NW
CHUNK = 128
N_CHUNKS = B_PER_W // CHUNK
NBUF = 13
N_GROUPS = N_CHUNKS // NBUF
OUT_SUBROWS = ROWS * 32 * 128 // DIM


def _emb_body(idx_hbm, qidx_hbm, table_hbm, out_hbm, idx_v, qidx_v, rows_v,
              *sems):
    gsems = sems[:NBUF]
    wsems = sems[NBUF:]
    wid = lax.axis_index("s") * NC + lax.axis_index("c")

    pltpu.sync_copy(idx_hbm.at[wid], idx_v)
    pltpu.sync_copy(qidx_hbm.at[wid], qidx_v)

    def gather(j, b):
        pltpu.make_async_copy(
            table_hbm.at[idx_v.at[j]], rows_v.at[b], gsems[b]
        ).start()

    def gather_wait(j, b):
        pltpu.make_async_copy(
            table_hbm.at[idx_v.at[j]], rows_v.at[b], gsems[b]
        ).wait()

    def write(j, b):
        pltpu.make_async_copy(
            rows_v.at[b], out_hbm.at[qidx_v.at[j]], wsems[b]
        ).start()

    def write_wait(j, b):
        pltpu.make_async_copy(
            rows_v.at[b], out_hbm.at[qidx_v.at[j]], wsems[b]
        ).wait()

    for b in range(NBUF - 1):
        gather(b, b)

    def body(g, carry):
        j0 = g * NBUF
        for b in range(NBUF):
            j = j0 + b
            gather_wait(j, b)
            write(j, b)
            bp = (b - 1) % NBUF
            jn = j + NBUF - 1

            if b == 0:
                @pl.when(j >= 1)
                def _():
                    write_wait(j - 1, bp)

                gather(jn, bp)
            else:
                @pl.when(jn < N_CHUNKS)
                def _():
                    write_wait(j - 1, bp)
                    gather(jn, bp)

        return carry

    lax.fori_loop(0, N_GROUPS, body, 0)

    for b in range(NBUF):
        write_wait(N_CHUNKS - NBUF + b, b)


_T = np.arange(B_TOTAL, dtype=np.int64)
_QIDX3 = ((_T // COLS) * 128 + (_T % COLS) * (128 // DIM)).astype(
    np.int32).reshape(NW, N_CHUNKS, CHUNK)


def kernel(input, weight):
    idx = input.reshape(-1).astype(jnp.int32)
    idx3 = idx.reshape(NW, N_CHUNKS, CHUNK)
    qidx3 = jnp.asarray(_QIDX3)

    mesh = plsc.VectorSubcoreMesh(core_axis_name="c", subcore_axis_name="s")
    run = pl.kernel(
        _emb_body,
        out_type=jax.ShapeDtypeStruct((OUT_SUBROWS, DIM), jnp.float32),
        mesh=mesh,
        scratch_types=[
            pltpu.VMEM((N_CHUNKS, CHUNK), jnp.int32),
            pltpu.VMEM((N_CHUNKS, CHUNK), jnp.int32),
            pltpu.VMEM((NBUF, CHUNK, DIM), jnp.float32),
        ]
        + [pltpu.SemaphoreType.DMA] * (2 * NBUF),
        compiler_params=pltpu.CompilerParams(use_tc_tiling_on_sc=False),
    )
    out = run(idx3, qidx3, weight)
    return out.reshape(ROWS, 32, 128)[:, :COLS, :DIM]
